# Initial kernel scaffold; baseline (speedup 1.0000x reference)
import jax
import jax.numpy as jnp
from jax.experimental import pallas as pl


def kernel(x, edge_index, batch, num_graphs, Wa1, ba1, Wa2, ba2, Wx1, bx1, Wx2, bx2, Wlin, blin):
    raise NotImplementedError("write your pallas kernel here")


# speedup vs baseline: 19.4372x; 19.4372x over previous
import functools

import jax
import jax.numpy as jnp
from jax import lax
from jax.experimental import pallas as pl
from jax.experimental.pallas import tpu as pltpu
from jax.experimental.pallas import tpu_sc as plsc

N = 10000
E = 320000
G = 64
D1CAT = 128
D2CAT = 64
DOUT = 10
FLAT = 32 * 32

NCORES = 2
NSUB = 16
E_PER_CORE = E // NCORES
E_PER_TILE = E_PER_CORE // NSUB
KCH = 80
NCHUNK = E_PER_TILE // KCH

def _row_range(s):
    r0 = s * 640
    nr = jnp.where(s == NSUB - 1, N - 640 * (NSUB - 1), 640)
    return r0, nr


@functools.cache
def _get_sc_degree():
    mesh = plsc.VectorSubcoreMesh(core_axis_name="c", subcore_axis_name="s")

    @functools.partial(
        pl.kernel,
        out_type=jax.ShapeDtypeStruct((NCORES, N), jnp.float32),
        mesh=mesh,
        scratch_types=[
            pltpu.VMEM((KCH,), jnp.int32),
            pltpu.VMEM((KCH,), jnp.float32),
            pltpu.VMEM_SHARED((N,), jnp.float32),
        ],
    )
    def _sc_degree(dst_hbm, zeros_hbm, out_hbm, idx_v, ones_v, acc_sh):
        c = lax.axis_index("c")
        s = lax.axis_index("s")

        for i in range(KCH // 16):
            ones_v[pl.ds(16 * i, 16)] = jnp.ones((16,), jnp.float32)

        @pl.when(s == 0)
        def _zero():
            pltpu.sync_copy(zeros_hbm, acc_sh)

        plsc.subcore_barrier()

        base = c * E_PER_CORE + s * E_PER_TILE

        def body(j, _):
            pltpu.sync_copy(dst_hbm.at[pl.ds(base + j * KCH, KCH)], idx_v)
            pltpu.sync_copy(ones_v, acc_sh.at[idx_v], add=True)
            return _

        lax.fori_loop(0, NCHUNK, body, None)
        plsc.subcore_barrier()

        @pl.when(s == 0)
        def _out():
            pltpu.sync_copy(acc_sh, out_hbm.at[c])

    return _sc_degree


@functools.cache
def _make_sc_prop(d):
    mesh = plsc.VectorSubcoreMesh(core_axis_name="c", subcore_axis_name="s")

    @functools.partial(
        pl.kernel,
        out_type=jax.ShapeDtypeStruct((NCORES, N, d), jnp.float32),
        mesh=mesh,
        compiler_params=pltpu.CompilerParams(use_tc_tiling_on_sc=(d == 128)),
        scratch_types=[
            pltpu.VMEM((KCH,), jnp.int32),
            pltpu.VMEM((KCH,), jnp.int32),
            pltpu.VMEM((KCH, d), jnp.float32),
            pltpu.SemaphoreType.DMA,
            pltpu.VMEM_SHARED((N, d), jnp.float32),
        ],
    )
    def _sc_prop(h_hbm, src_hbm, dst_hbm, zeros_hbm, out_hbm,
                 src_v, dst_v, rows_v, sem, acc_sh):
        c = lax.axis_index("c")
        s = lax.axis_index("s")

        r0, nr = _row_range(s)
        pltpu.sync_copy(zeros_hbm.at[pl.ds(r0, nr)], acc_sh.at[pl.ds(r0, nr)])
        plsc.subcore_barrier()

        base = c * E_PER_CORE + s * E_PER_TILE

        def body(j, _):
            e0 = base + j * KCH
            pltpu.sync_copy(src_hbm.at[pl.ds(e0, KCH)], src_v)
            pltpu.sync_copy(dst_hbm.at[pl.ds(e0, KCH)], dst_v)
            pltpu.async_copy(h_hbm.at[src_v], rows_v, sem).wait()
            pltpu.sync_copy(rows_v, acc_sh.at[dst_v], add=True)
            return _

        lax.fori_loop(0, NCHUNK, body, None)
        plsc.subcore_barrier()
        pltpu.sync_copy(acc_sh.at[pl.ds(r0, nr)], out_hbm.at[c, pl.ds(r0, nr)])

    return _sc_prop


def _tc1_body(degp_ref, x_ref, w_ref, dis_ref, h1s_ref):
    dp = degp_ref[...]
    dis = lax.rsqrt(dp[0] + dp[1] + 2.0)
    h = jnp.dot(x_ref[...], w_ref[...], preferred_element_type=jnp.float32)
    dis_ref[...] = dis
    h1s_ref[...] = dis * h


def _tc1(degp, x, wcat1):
    return pl.pallas_call(
        _tc1_body,
        out_shape=(
            jax.ShapeDtypeStruct((N, 1), jnp.float32),
            jax.ShapeDtypeStruct((N, D1CAT), jnp.float32),
        ),
    )(degp, x, wcat1)


def _tc2_body(accp_ref, dis_ref, h1s_ref, b_ref, w_ref, h2s_ref):
    ap = accp_ref[...]
    dis = dis_ref[...]
    acc = ap[0] + ap[1] + 2.0 * h1s_ref[...]
    y1 = jnp.maximum(dis * acc + b_ref[...], 0.0)
    h2 = jnp.dot(y1, w_ref[...], preferred_element_type=jnp.float32)
    h2s_ref[...] = dis * h2


def _tc2(accp, dis, h1s, bcat1, wblk):
    return pl.pallas_call(
        _tc2_body,
        out_shape=jax.ShapeDtypeStruct((N, D2CAT), jnp.float32),
    )(accp, dis, h1s, bcat1, wblk)


_BB = 400
_NB = N // _BB


def _dot_t(a, b):
    return lax.dot_general(a, b, (((0,), (0,)), ((), ())),
                           preferred_element_type=jnp.float32)


def _tc3_body(accp_ref, dis_ref, h2s_ref, b_ref, bcol_ref,
              gmask_ref, wlin_ref, blin_ref, out_ref, pre_ref, xx_ref):
    ap = accp_ref[...]
    dis = dis_ref[...]
    o2 = dis * (ap[0] + ap[1] + 2.0 * h2s_ref[...]) + b_ref[...]

    g_row = lax.broadcasted_iota(jnp.int32, (1, G), 1)

    jj = lax.broadcasted_iota(jnp.int32, (32, FLAT), 1)
    rr = lax.broadcasted_iota(jnp.int32, (32, FLAT), 0)
    k_rep = (jj // 32 == rr).astype(jnp.float32)
    k_tile = (jj % 32 == rr).astype(jnp.float32)

    r64 = lax.broadcasted_iota(jnp.int32, (D2CAT, 32), 0)
    c32 = lax.broadcasted_iota(jnp.int32, (D2CAT, 32), 1)
    e1 = (r64 == c32).astype(jnp.float32)
    e2 = (r64 == c32 + 32).astype(jnp.float32)

    pre_ref[...] = jnp.dot(o2, e1, preferred_element_type=jnp.float32)
    xx_ref[...] = jnp.maximum(
        jnp.dot(o2, e2, preferred_element_type=jnp.float32), 0.0)

    bcol_full = bcol_ref[...]
    pre_full = pre_ref[...]
    rows = [
        jnp.max(jnp.where(bcol_full == g, pre_full, -jnp.inf),
                axis=0, keepdims=True)
        for g in range(G)
    ]
    m = jnp.concatenate(rows, axis=0)
    m = jnp.where(m == -jnp.inf, 0.0, m)

    def accbody(j, carry):
        s_acc, p_acc = carry
        pre_b = pre_ref[pl.ds(j * _BB, _BB), :]
        xx_b = xx_ref[pl.ds(j * _BB, _BB), :]
        s_ng = (bcol_ref[pl.ds(j * _BB, _BB), :] == g_row
                ).astype(jnp.float32)
        m_n = jnp.dot(s_ng, m, preferred_element_type=jnp.float32)
        e = jnp.exp(pre_b - m_n)
        s_acc = s_acc + _dot_t(s_ng, e)
        ek = jnp.dot(e, k_rep, preferred_element_type=jnp.float32)
        xt = jnp.dot(xx_b, k_tile, preferred_element_type=jnp.float32)
        p_acc = p_acc + _dot_t(s_ng, ek * xt)
        return s_acc, p_acc

    s_sum, p_sum = lax.fori_loop(
        0, _NB, accbody,
        (jnp.zeros((G, 32), jnp.float32), jnp.zeros((G, FLAT), jnp.float32)))

    srep = jnp.dot(s_sum, k_rep, preferred_element_type=jnp.float32)
    flat = p_sum / (srep + 1e-16) * gmask_ref[...]
    logits = jnp.dot(flat, wlin_ref[...],
                     preferred_element_type=jnp.float32) + blin_ref[...]
    z = logits - jnp.max(logits, axis=-1, keepdims=True)
    ez = jnp.exp(z)
    out_ref[...] = ez / jnp.sum(ez, axis=-1, keepdims=True)


def _tc3(accp, dis, h2s, bcat2, bcol, gmask, wlin_t, blin):
    return pl.pallas_call(
        _tc3_body,
        out_shape=jax.ShapeDtypeStruct((G, DOUT), jnp.float32),
        scratch_shapes=[pltpu.VMEM((N, 32), jnp.float32),
                        pltpu.VMEM((N, 32), jnp.float32)],
    )(accp, dis, h2s, bcat2, bcol, gmask, wlin_t, blin)


def kernel(x, edge_index, batch, num_graphs,
           Wa1, ba1, Wa2, ba2, Wx1, bx1, Wx2, bx2, Wlin, blin):
    src = edge_index[0].astype(jnp.int32)
    dst = edge_index[1].astype(jnp.int32)
    b32 = batch.astype(jnp.int32)

    wcat1 = jnp.concatenate([Wa1, Wx1], axis=1)
    bcat1 = jnp.concatenate([ba1, bx1])[None, :]
    wblk = jnp.zeros((D1CAT, D2CAT), jnp.float32)
    wblk = wblk.at[:64, :32].set(Wa2).at[64:, 32:].set(Wx2)
    bcat2 = jnp.concatenate([ba2, bx2])[None, :]
    gmask = (jnp.arange(G) < num_graphs).astype(jnp.float32)[:, None]

    zeros_n = jnp.zeros((N,), jnp.float32)
    zeros1 = jnp.zeros((N, D1CAT), jnp.float32)
    zeros2 = jnp.zeros((N, D2CAT), jnp.float32)

    degp = _get_sc_degree()(dst, zeros_n)
    dis, h1s = _tc1(degp.reshape(NCORES, N, 1), x, wcat1)
    acc1 = _make_sc_prop(D1CAT)(h1s, src, dst, zeros1)
    h2s = _tc2(acc1, dis, h1s, bcat1, wblk)
    acc2 = _make_sc_prop(D2CAT)(h2s, src, dst, zeros2)
    out = _tc3(acc2, dis, h2s, bcat2,
               b32[:, None], gmask, Wlin.T, blin[None, :])
    return out


# speedup vs baseline: 26.7102x; 1.3742x over previous
import functools

import jax
import jax.numpy as jnp
from jax import lax
from jax.experimental import pallas as pl
from jax.experimental.pallas import tpu as pltpu
from jax.experimental.pallas import tpu_sc as plsc

N = 10000
E = 320000
G = 64
D1CAT = 128
D2CAT = 64
DOUT = 10
FLAT = 32 * 32

NCORES = 2
NSUB = 16
E_PER_CORE = E // NCORES
E_PER_TILE = E_PER_CORE // NSUB
KCH = 80
NCHUNK = E_PER_TILE // KCH

def _row_range(s):
    r0 = s * 640
    nr = jnp.where(s == NSUB - 1, N - 640 * (NSUB - 1), 640)
    return r0, nr


@functools.cache
def _get_sc_degree():
    mesh = plsc.VectorSubcoreMesh(core_axis_name="c", subcore_axis_name="s")

    @functools.partial(
        pl.kernel,
        out_type=jax.ShapeDtypeStruct((NCORES, N), jnp.float32),
        mesh=mesh,
        scratch_types=[
            pltpu.VMEM((KCH,), jnp.int32),
            pltpu.VMEM((KCH,), jnp.float32),
            pltpu.VMEM_SHARED((N,), jnp.float32),
        ],
    )
    def _sc_degree(dst_hbm, zeros_hbm, out_hbm, idx_v, ones_v, acc_sh):
        c = lax.axis_index("c")
        s = lax.axis_index("s")

        for i in range(KCH // 16):
            ones_v[pl.ds(16 * i, 16)] = jnp.ones((16,), jnp.float32)

        @pl.when(s == 0)
        def _zero():
            pltpu.sync_copy(zeros_hbm, acc_sh)

        plsc.subcore_barrier()

        base = c * E_PER_CORE + s * E_PER_TILE

        def body(j, _):
            pltpu.sync_copy(dst_hbm.at[pl.ds(base + j * KCH, KCH)], idx_v)
            pltpu.sync_copy(ones_v, acc_sh.at[idx_v], add=True)
            return _

        lax.fori_loop(0, NCHUNK, body, None)
        plsc.subcore_barrier()

        @pl.when(s == 0)
        def _out():
            pltpu.sync_copy(acc_sh, out_hbm.at[c])

    return _sc_degree


@functools.cache
def _make_sc_prop(d):
    mesh = plsc.VectorSubcoreMesh(core_axis_name="c", subcore_axis_name="s")

    @functools.partial(
        pl.kernel,
        out_type=jax.ShapeDtypeStruct((NCORES, N, d), jnp.float32),
        mesh=mesh,
        compiler_params=pltpu.CompilerParams(use_tc_tiling_on_sc=(d == 128)),
        scratch_types=[
            pltpu.VMEM((KCH,), jnp.int32),
            pltpu.VMEM((KCH,), jnp.int32),
            pltpu.VMEM((KCH,), jnp.int32),
            pltpu.VMEM((KCH,), jnp.int32),
            pltpu.VMEM((KCH, d), jnp.float32),
            pltpu.VMEM((KCH, d), jnp.float32),
            pltpu.SemaphoreType.DMA,
            pltpu.SemaphoreType.DMA,
            pltpu.VMEM_SHARED((N, d), jnp.float32),
        ],
    )
    def _sc_prop(h_hbm, src_hbm, dst_hbm, zeros_hbm, out_hbm,
                 src0, src1, dst0, dst1, rows0, rows1, sem0, sem1, acc_sh):
        c = lax.axis_index("c")
        s = lax.axis_index("s")
        bufs = ((src0, dst0, rows0, sem0), (src1, dst1, rows1, sem1))

        base = c * E_PER_CORE + s * E_PER_TILE

        def issue(j, b):
            sv, dv, rv, sm = bufs[b]
            e0 = base + j * KCH
            pltpu.sync_copy(src_hbm.at[pl.ds(e0, KCH)], sv)
            pltpu.sync_copy(dst_hbm.at[pl.ds(e0, KCH)], dv)
            pltpu.async_copy(h_hbm.at[sv], rv, sm)

        issue(0, 0)
        issue(1, 1)
        r0, nr = _row_range(s)
        pltpu.sync_copy(zeros_hbm.at[pl.ds(r0, nr)], acc_sh.at[pl.ds(r0, nr)])
        plsc.subcore_barrier()

        def pair(i, _):
            j0 = 2 * i
            for b in range(2):
                sv, dv, rv, sm = bufs[b]
                pltpu.make_async_copy(h_hbm.at[sv], rv, sm).wait()
                pltpu.sync_copy(rv, acc_sh.at[dv], add=True)
                nxt = j0 + b + 2

                @pl.when(nxt < NCHUNK)
                def _pf():
                    issue(nxt, b)
            return _

        lax.fori_loop(0, NCHUNK // 2, pair, None)
        if NCHUNK % 2:
            sv, dv, rv, sm = bufs[0]
            pltpu.make_async_copy(h_hbm.at[sv], rv, sm).wait()
            pltpu.sync_copy(rv, acc_sh.at[dv], add=True)
        plsc.subcore_barrier()
        pltpu.sync_copy(acc_sh.at[pl.ds(r0, nr)], out_hbm.at[c, pl.ds(r0, nr)])

    return _sc_prop


def _tc1_body(degp_ref, x_ref, w_ref, dis_ref, h1s_ref):
    dp = degp_ref[...]
    dis = lax.rsqrt(dp[0] + dp[1] + 2.0)
    h = jnp.dot(x_ref[...], w_ref[...], preferred_element_type=jnp.float32)
    dis_ref[...] = dis
    h1s_ref[...] = dis * h


def _tc1(degp, x, wcat1):
    return pl.pallas_call(
        _tc1_body,
        out_shape=(
            jax.ShapeDtypeStruct((N, 1), jnp.float32),
            jax.ShapeDtypeStruct((N, D1CAT), jnp.float32),
        ),
    )(degp, x, wcat1)


def _tc2_body(accp_ref, dis_ref, h1s_ref, b_ref, w_ref, h2s_ref):
    ap = accp_ref[...]
    dis = dis_ref[...]
    acc = ap[0] + ap[1] + 2.0 * h1s_ref[...]
    y1 = jnp.maximum(dis * acc + b_ref[...], 0.0)
    h2 = jnp.dot(y1, w_ref[...], preferred_element_type=jnp.float32)
    h2s_ref[...] = dis * h2


def _tc2(accp, dis, h1s, bcat1, wblk):
    return pl.pallas_call(
        _tc2_body,
        out_shape=jax.ShapeDtypeStruct((N, D2CAT), jnp.float32),
    )(accp, dis, h1s, bcat1, wblk)


_BB = 400
_NB = N // _BB


def _dot_t(a, b):
    return lax.dot_general(a, b, (((0,), (0,)), ((), ())),
                           preferred_element_type=jnp.float32)


def _tc3_body(accp_ref, dis_ref, h2s_ref, b_ref, bcol_ref,
              gmask_ref, wlin_ref, blin_ref, out_ref, pre_ref, xx_ref):
    ap = accp_ref[...]
    dis = dis_ref[...]
    o2 = dis * (ap[0] + ap[1] + 2.0 * h2s_ref[...]) + b_ref[...]

    g_row = lax.broadcasted_iota(jnp.int32, (1, G), 1)

    jj = lax.broadcasted_iota(jnp.int32, (32, FLAT), 1)
    rr = lax.broadcasted_iota(jnp.int32, (32, FLAT), 0)
    k_rep = (jj // 32 == rr).astype(jnp.float32)
    k_tile = (jj % 32 == rr).astype(jnp.float32)

    r64 = lax.broadcasted_iota(jnp.int32, (D2CAT, 32), 0)
    c32 = lax.broadcasted_iota(jnp.int32, (D2CAT, 32), 1)
    e1 = (r64 == c32).astype(jnp.float32)
    e2 = (r64 == c32 + 32).astype(jnp.float32)

    pre_ref[...] = jnp.dot(o2, e1, preferred_element_type=jnp.float32)
    xx_ref[...] = jnp.maximum(
        jnp.dot(o2, e2, preferred_element_type=jnp.float32), 0.0)

    bcol_full = bcol_ref[...]
    pre_full = pre_ref[...]
    rows = [
        jnp.max(jnp.where(bcol_full == g, pre_full, -jnp.inf),
                axis=0, keepdims=True)
        for g in range(G)
    ]
    m = jnp.concatenate(rows, axis=0)
    m = jnp.where(m == -jnp.inf, 0.0, m)

    def accbody(j, carry):
        s_acc, p_acc = carry
        pre_b = pre_ref[pl.ds(j * _BB, _BB), :]
        xx_b = xx_ref[pl.ds(j * _BB, _BB), :]
        s_ng = (bcol_ref[pl.ds(j * _BB, _BB), :] == g_row
                ).astype(jnp.float32)
        m_n = jnp.dot(s_ng, m, preferred_element_type=jnp.float32)
        e = jnp.exp(pre_b - m_n)
        s_acc = s_acc + _dot_t(s_ng, e)
        ek = jnp.dot(e, k_rep, preferred_element_type=jnp.float32)
        xt = jnp.dot(xx_b, k_tile, preferred_element_type=jnp.float32)
        p_acc = p_acc + _dot_t(s_ng, ek * xt)
        return s_acc, p_acc

    s_sum, p_sum = lax.fori_loop(
        0, _NB, accbody,
        (jnp.zeros((G, 32), jnp.float32), jnp.zeros((G, FLAT), jnp.float32)))

    srep = jnp.dot(s_sum, k_rep, preferred_element_type=jnp.float32)
    flat = p_sum / (srep + 1e-16) * gmask_ref[...]
    logits = jnp.dot(flat, wlin_ref[...],
                     preferred_element_type=jnp.float32) + blin_ref[...]
    z = logits - jnp.max(logits, axis=-1, keepdims=True)
    ez = jnp.exp(z)
    out_ref[...] = ez / jnp.sum(ez, axis=-1, keepdims=True)


def _tc3(accp, dis, h2s, bcat2, bcol, gmask, wlin_t, blin):
    return pl.pallas_call(
        _tc3_body,
        out_shape=jax.ShapeDtypeStruct((G, DOUT), jnp.float32),
        scratch_shapes=[pltpu.VMEM((N, 32), jnp.float32),
                        pltpu.VMEM((N, 32), jnp.float32)],
    )(accp, dis, h2s, bcat2, bcol, gmask, wlin_t, blin)


def kernel(x, edge_index, batch, num_graphs,
           Wa1, ba1, Wa2, ba2, Wx1, bx1, Wx2, bx2, Wlin, blin):
    src = edge_index[0].astype(jnp.int32)
    dst = edge_index[1].astype(jnp.int32)
    b32 = batch.astype(jnp.int32)

    wcat1 = jnp.concatenate([Wa1, Wx1], axis=1)
    bcat1 = jnp.concatenate([ba1, bx1])[None, :]
    wblk = jnp.zeros((D1CAT, D2CAT), jnp.float32)
    wblk = wblk.at[:64, :32].set(Wa2).at[64:, 32:].set(Wx2)
    bcat2 = jnp.concatenate([ba2, bx2])[None, :]
    gmask = (jnp.arange(G) < num_graphs).astype(jnp.float32)[:, None]

    zeros_n = jnp.zeros((N,), jnp.float32)
    zeros1 = jnp.zeros((N, D1CAT), jnp.float32)
    zeros2 = jnp.zeros((N, D2CAT), jnp.float32)

    degp = _get_sc_degree()(dst, zeros_n)
    dis, h1s = _tc1(degp.reshape(NCORES, N, 1), x, wcat1)
    acc1 = _make_sc_prop(D1CAT)(h1s, src, dst, zeros1)
    h2s = _tc2(acc1, dis, h1s, bcat1, wblk)
    acc2 = _make_sc_prop(D2CAT)(h2s, src, dst, zeros2)
    out = _tc3(acc2, dis, h2s, bcat2,
               b32[:, None], gmask, Wlin.T, blin[None, :])
    return out


# speedup vs baseline: 28.1964x; 1.0556x over previous
import functools

import jax
import jax.numpy as jnp
from jax import lax
from jax.experimental import pallas as pl
from jax.experimental.pallas import tpu as pltpu
from jax.experimental.pallas import tpu_sc as plsc

N = 10000
E = 320000
G = 64
D1CAT = 128
D2CAT = 64
DOUT = 10
FLAT = 32 * 32

NCORES = 2
NSUB = 16
E_PER_CORE = E // NCORES
E_PER_TILE = E_PER_CORE // NSUB
KCH = 80
NCHUNK = E_PER_TILE // KCH

def _row_range(s):
    r0 = s * 640
    nr = jnp.where(s == NSUB - 1, N - 640 * (NSUB - 1), 640)
    return r0, nr


@functools.cache
def _get_sc_degree():
    mesh = plsc.VectorSubcoreMesh(core_axis_name="c", subcore_axis_name="s")

    @functools.partial(
        pl.kernel,
        out_type=jax.ShapeDtypeStruct((NCORES, N), jnp.float32),
        mesh=mesh,
        scratch_types=[
            pltpu.VMEM((KCH,), jnp.int32),
            pltpu.VMEM((KCH,), jnp.int32),
            pltpu.VMEM((KCH,), jnp.float32),
            pltpu.SemaphoreType.DMA,
            pltpu.SemaphoreType.DMA,
            pltpu.VMEM_SHARED((N,), jnp.float32),
        ],
    )
    def _sc_degree(dst_hbm, zeros_hbm, out_hbm, idx0, idx1, ones_v,
                   sem0, sem1, acc_sh):
        c = lax.axis_index("c")
        s = lax.axis_index("s")
        bufs = ((idx0, sem0), (idx1, sem1))

        base = c * E_PER_CORE + s * E_PER_TILE

        def issue(j, b):
            iv, sm = bufs[b]
            pltpu.async_copy(dst_hbm.at[pl.ds(base + j * KCH, KCH)], iv, sm)

        issue(0, 0)
        issue(1, 1)

        for i in range(KCH // 16):
            ones_v[pl.ds(16 * i, 16)] = jnp.ones((16,), jnp.float32)

        @pl.when(s == 0)
        def _zero():
            pltpu.sync_copy(zeros_hbm, acc_sh)

        plsc.subcore_barrier()

        def pair(i, _):
            j0 = 2 * i
            for b in range(2):
                iv, sm = bufs[b]
                pltpu.make_async_copy(dst_hbm.at[pl.ds(base, KCH)], iv, sm).wait()
                pltpu.sync_copy(ones_v, acc_sh.at[iv], add=True)
                nxt = j0 + b + 2

                @pl.when(nxt < NCHUNK)
                def _pf():
                    issue(nxt, b)
            return _

        lax.fori_loop(0, NCHUNK // 2, pair, None)
        if NCHUNK % 2:
            iv, sm = bufs[0]
            pltpu.make_async_copy(dst_hbm.at[pl.ds(base, KCH)], iv, sm).wait()
            pltpu.sync_copy(ones_v, acc_sh.at[iv], add=True)
        plsc.subcore_barrier()

        @pl.when(s == 0)
        def _out():
            pltpu.sync_copy(acc_sh, out_hbm.at[c])

    return _sc_degree


@functools.cache
def _make_sc_prop(d):
    mesh = plsc.VectorSubcoreMesh(core_axis_name="c", subcore_axis_name="s")

    @functools.partial(
        pl.kernel,
        out_type=jax.ShapeDtypeStruct((NCORES, N, d), jnp.float32),
        mesh=mesh,
        compiler_params=pltpu.CompilerParams(use_tc_tiling_on_sc=(d == 128)),
        scratch_types=[
            pltpu.VMEM((KCH,), jnp.int32),
            pltpu.VMEM((KCH,), jnp.int32),
            pltpu.VMEM((KCH,), jnp.int32),
            pltpu.VMEM((KCH,), jnp.int32),
            pltpu.VMEM((KCH, d), jnp.float32),
            pltpu.VMEM((KCH, d), jnp.float32),
            pltpu.SemaphoreType.DMA,
            pltpu.SemaphoreType.DMA,
            pltpu.VMEM_SHARED((N, d), jnp.float32),
        ],
    )
    def _sc_prop(h_hbm, src_hbm, dst_hbm, zeros_hbm, out_hbm,
                 src0, src1, dst0, dst1, rows0, rows1, sem0, sem1, acc_sh):
        c = lax.axis_index("c")
        s = lax.axis_index("s")
        bufs = ((src0, dst0, rows0, sem0), (src1, dst1, rows1, sem1))

        base = c * E_PER_CORE + s * E_PER_TILE

        def issue(j, b):
            sv, dv, rv, sm = bufs[b]
            e0 = base + j * KCH
            pltpu.sync_copy(src_hbm.at[pl.ds(e0, KCH)], sv)
            pltpu.sync_copy(dst_hbm.at[pl.ds(e0, KCH)], dv)
            pltpu.async_copy(h_hbm.at[sv], rv, sm)

        issue(0, 0)
        issue(1, 1)
        r0, nr = _row_range(s)
        pltpu.sync_copy(zeros_hbm.at[pl.ds(r0, nr)], acc_sh.at[pl.ds(r0, nr)])
        plsc.subcore_barrier()

        def pair(i, _):
            j0 = 2 * i
            for b in range(2):
                sv, dv, rv, sm = bufs[b]
                pltpu.make_async_copy(h_hbm.at[sv], rv, sm).wait()
                pltpu.sync_copy(rv, acc_sh.at[dv], add=True)
                nxt = j0 + b + 2

                @pl.when(nxt < NCHUNK)
                def _pf():
                    issue(nxt, b)
            return _

        lax.fori_loop(0, NCHUNK // 2, pair, None)
        if NCHUNK % 2:
            sv, dv, rv, sm = bufs[0]
            pltpu.make_async_copy(h_hbm.at[sv], rv, sm).wait()
            pltpu.sync_copy(rv, acc_sh.at[dv], add=True)
        plsc.subcore_barrier()
        pltpu.sync_copy(acc_sh.at[pl.ds(r0, nr)], out_hbm.at[c, pl.ds(r0, nr)])

    return _sc_prop


def _tc1_body(degp_ref, x_ref, w_ref, dis_ref, h1s_ref):
    dp = degp_ref[...]
    dis = lax.rsqrt(dp[0] + dp[1] + 2.0)
    h = jnp.dot(x_ref[...], w_ref[...], preferred_element_type=jnp.float32)
    dis_ref[...] = dis
    h1s_ref[...] = dis * h


def _tc1(degp, x, wcat1):
    return pl.pallas_call(
        _tc1_body,
        out_shape=(
            jax.ShapeDtypeStruct((N, 1), jnp.float32),
            jax.ShapeDtypeStruct((N, D1CAT), jnp.float32),
        ),
    )(degp, x, wcat1)


def _tc2_body(accp_ref, dis_ref, h1s_ref, b_ref, w_ref, h2s_ref):
    ap = accp_ref[...]
    dis = dis_ref[...]
    acc = ap[0] + ap[1] + 2.0 * h1s_ref[...]
    y1 = jnp.maximum(dis * acc + b_ref[...], 0.0)
    h2 = jnp.dot(y1, w_ref[...], preferred_element_type=jnp.float32)
    h2s_ref[...] = dis * h2


def _tc2(accp, dis, h1s, bcat1, wblk):
    return pl.pallas_call(
        _tc2_body,
        out_shape=jax.ShapeDtypeStruct((N, D2CAT), jnp.float32),
    )(accp, dis, h1s, bcat1, wblk)


_BB = 400
_NB = N // _BB


def _dot_t(a, b):
    return lax.dot_general(a, b, (((0,), (0,)), ((), ())),
                           preferred_element_type=jnp.float32)


def _tc3_body(accp_ref, dis_ref, h2s_ref, b_ref, bcol_ref,
              gmask_ref, wlin_ref, blin_ref, out_ref, pre_ref, xx_ref):
    ap = accp_ref[...]
    dis = dis_ref[...]
    o2 = dis * (ap[0] + ap[1] + 2.0 * h2s_ref[...]) + b_ref[...]

    g_row = lax.broadcasted_iota(jnp.int32, (1, G), 1)

    jj = lax.broadcasted_iota(jnp.int32, (32, FLAT), 1)
    rr = lax.broadcasted_iota(jnp.int32, (32, FLAT), 0)
    k_rep = (jj // 32 == rr).astype(jnp.float32)
    k_tile = (jj % 32 == rr).astype(jnp.float32)

    r64 = lax.broadcasted_iota(jnp.int32, (D2CAT, 32), 0)
    c32 = lax.broadcasted_iota(jnp.int32, (D2CAT, 32), 1)
    e1 = (r64 == c32).astype(jnp.float32)
    e2 = (r64 == c32 + 32).astype(jnp.float32)

    pre_ref[...] = jnp.dot(o2, e1, preferred_element_type=jnp.float32)
    xx_ref[...] = jnp.maximum(
        jnp.dot(o2, e2, preferred_element_type=jnp.float32), 0.0)

    bcol_full = bcol_ref[...]
    pre_full = pre_ref[...]
    rows = [
        jnp.max(jnp.where(bcol_full == g, pre_full, -jnp.inf),
                axis=0, keepdims=True)
        for g in range(G)
    ]
    m = jnp.concatenate(rows, axis=0)
    m = jnp.where(m == -jnp.inf, 0.0, m)

    def accbody(j, carry):
        s_acc, p_acc = carry
        pre_b = pre_ref[pl.ds(j * _BB, _BB), :]
        xx_b = xx_ref[pl.ds(j * _BB, _BB), :]
        s_ng = (bcol_ref[pl.ds(j * _BB, _BB), :] == g_row
                ).astype(jnp.float32)
        m_n = jnp.dot(s_ng, m, preferred_element_type=jnp.float32)
        e = jnp.exp(pre_b - m_n)
        s_acc = s_acc + _dot_t(s_ng, e)
        ek = jnp.dot(e, k_rep, preferred_element_type=jnp.float32)
        xt = jnp.dot(xx_b, k_tile, preferred_element_type=jnp.float32)
        p_acc = p_acc + _dot_t(s_ng, ek * xt)
        return s_acc, p_acc

    s_sum, p_sum = lax.fori_loop(
        0, _NB, accbody,
        (jnp.zeros((G, 32), jnp.float32), jnp.zeros((G, FLAT), jnp.float32)))

    srep = jnp.dot(s_sum, k_rep, preferred_element_type=jnp.float32)
    flat = p_sum / (srep + 1e-16) * gmask_ref[...]
    logits = jnp.dot(flat, wlin_ref[...],
                     preferred_element_type=jnp.float32) + blin_ref[...]
    z = logits - jnp.max(logits, axis=-1, keepdims=True)
    ez = jnp.exp(z)
    out_ref[...] = ez / jnp.sum(ez, axis=-1, keepdims=True)


def _tc3(accp, dis, h2s, bcat2, bcol, gmask, wlin_t, blin):
    return pl.pallas_call(
        _tc3_body,
        out_shape=jax.ShapeDtypeStruct((G, DOUT), jnp.float32),
        scratch_shapes=[pltpu.VMEM((N, 32), jnp.float32),
                        pltpu.VMEM((N, 32), jnp.float32)],
    )(accp, dis, h2s, bcat2, bcol, gmask, wlin_t, blin)


def kernel(x, edge_index, batch, num_graphs,
           Wa1, ba1, Wa2, ba2, Wx1, bx1, Wx2, bx2, Wlin, blin):
    src = edge_index[0].astype(jnp.int32)
    dst = edge_index[1].astype(jnp.int32)
    b32 = batch.astype(jnp.int32)

    wcat1 = jnp.concatenate([Wa1, Wx1], axis=1)
    bcat1 = jnp.concatenate([ba1, bx1])[None, :]
    wblk = jnp.zeros((D1CAT, D2CAT), jnp.float32)
    wblk = wblk.at[:64, :32].set(Wa2).at[64:, 32:].set(Wx2)
    bcat2 = jnp.concatenate([ba2, bx2])[None, :]
    gmask = (jnp.arange(G) < num_graphs).astype(jnp.float32)[:, None]

    zeros_n = jnp.zeros((N,), jnp.float32)
    zeros1 = jnp.zeros((N, D1CAT), jnp.float32)
    zeros2 = jnp.zeros((N, D2CAT), jnp.float32)

    degp = _get_sc_degree()(dst, zeros_n)
    dis, h1s = _tc1(degp.reshape(NCORES, N, 1), x, wcat1)
    acc1 = _make_sc_prop(D1CAT)(h1s, src, dst, zeros1)
    h2s = _tc2(acc1, dis, h1s, bcat1, wblk)
    acc2 = _make_sc_prop(D2CAT)(h2s, src, dst, zeros2)
    out = _tc3(acc2, dis, h2s, bcat2,
               b32[:, None], gmask, Wlin.T, blin[None, :])
    return out


# speedup vs baseline: 28.2231x; 1.0009x over previous
import functools

import jax
import jax.numpy as jnp
from jax import lax
from jax.experimental import pallas as pl
from jax.experimental.pallas import tpu as pltpu
from jax.experimental.pallas import tpu_sc as plsc

N = 10000
E = 320000
G = 64
D1CAT = 128
D2CAT = 64
DOUT = 10
FLAT = 32 * 32

NCORES = 2
NSUB = 16
E_PER_CORE = E // NCORES
E_PER_TILE = E_PER_CORE // NSUB
KCH = 80
NCHUNK = E_PER_TILE // KCH
NBUF = 3

def _row_range(s):
    r0 = s * 640
    nr = jnp.where(s == NSUB - 1, N - 640 * (NSUB - 1), 640)
    return r0, nr


@functools.cache
def _get_sc_degree():
    mesh = plsc.VectorSubcoreMesh(core_axis_name="c", subcore_axis_name="s")

    @functools.partial(
        pl.kernel,
        out_type=jax.ShapeDtypeStruct((NCORES, N), jnp.float32),
        mesh=mesh,
        scratch_types=[
            pltpu.VMEM((KCH,), jnp.int32),
            pltpu.VMEM((KCH,), jnp.int32),
            pltpu.VMEM((KCH,), jnp.float32),
            pltpu.SemaphoreType.DMA,
            pltpu.SemaphoreType.DMA,
            pltpu.VMEM_SHARED((N,), jnp.float32),
        ],
    )
    def _sc_degree(dst_hbm, zeros_hbm, out_hbm, idx0, idx1, ones_v,
                   sem0, sem1, acc_sh):
        c = lax.axis_index("c")
        s = lax.axis_index("s")
        bufs = ((idx0, sem0), (idx1, sem1))

        base = c * E_PER_CORE + s * E_PER_TILE

        def issue(j, b):
            iv, sm = bufs[b]
            pltpu.async_copy(dst_hbm.at[pl.ds(base + j * KCH, KCH)], iv, sm)

        issue(0, 0)
        issue(1, 1)

        for i in range(KCH // 16):
            ones_v[pl.ds(16 * i, 16)] = jnp.ones((16,), jnp.float32)

        @pl.when(s == 0)
        def _zero():
            pltpu.sync_copy(zeros_hbm, acc_sh)

        plsc.subcore_barrier()

        def pair(i, _):
            j0 = 2 * i
            for b in range(2):
                iv, sm = bufs[b]
                pltpu.make_async_copy(dst_hbm.at[pl.ds(base, KCH)], iv, sm).wait()
                pltpu.sync_copy(ones_v, acc_sh.at[iv], add=True)
                nxt = j0 + b + 2

                @pl.when(nxt < NCHUNK)
                def _pf():
                    issue(nxt, b)
            return _

        lax.fori_loop(0, NCHUNK // 2, pair, None)
        if NCHUNK % 2:
            iv, sm = bufs[0]
            pltpu.make_async_copy(dst_hbm.at[pl.ds(base, KCH)], iv, sm).wait()
            pltpu.sync_copy(ones_v, acc_sh.at[iv], add=True)
        plsc.subcore_barrier()

        @pl.when(s == 0)
        def _out():
            pltpu.sync_copy(acc_sh, out_hbm.at[c])

    return _sc_degree


@functools.cache
def _make_sc_prop(d):
    mesh = plsc.VectorSubcoreMesh(core_axis_name="c", subcore_axis_name="s")

    @functools.partial(
        pl.kernel,
        out_type=jax.ShapeDtypeStruct((NCORES, N, d), jnp.float32),
        mesh=mesh,
        compiler_params=pltpu.CompilerParams(use_tc_tiling_on_sc=(d == 128)),
        scratch_types=(
            [pltpu.VMEM((KCH,), jnp.int32)] * (2 * NBUF)
            + [pltpu.VMEM((KCH, d), jnp.float32)] * NBUF
            + [pltpu.SemaphoreType.DMA] * NBUF
            + [pltpu.VMEM_SHARED((N, d), jnp.float32)]
        ),
    )
    def _sc_prop(h_hbm, src_hbm, dst_hbm, zeros_hbm, out_hbm, *scratch):
        c = lax.axis_index("c")
        s = lax.axis_index("s")
        srcs = scratch[0:NBUF]
        dsts = scratch[NBUF:2 * NBUF]
        rows = scratch[2 * NBUF:3 * NBUF]
        sems = scratch[3 * NBUF:4 * NBUF]
        acc_sh = scratch[4 * NBUF]
        bufs = tuple(zip(srcs, dsts, rows, sems))

        base = c * E_PER_CORE + s * E_PER_TILE

        def issue(j, b):
            sv, dv, rv, sm = bufs[b]
            e0 = base + j * KCH
            pltpu.sync_copy(src_hbm.at[pl.ds(e0, KCH)], sv)
            pltpu.sync_copy(dst_hbm.at[pl.ds(e0, KCH)], dv)
            pltpu.async_copy(h_hbm.at[sv], rv, sm)

        def drain_scatter(b):
            sv, dv, rv, sm = bufs[b]
            pltpu.make_async_copy(h_hbm.at[sv], rv, sm).wait()
            pltpu.sync_copy(rv, acc_sh.at[dv], add=True)

        for b in range(NBUF):
            issue(b, b)
        r0, nr = _row_range(s)
        pltpu.sync_copy(zeros_hbm.at[pl.ds(r0, nr)], acc_sh.at[pl.ds(r0, nr)])
        plsc.subcore_barrier()

        def group(i, _):
            j0 = NBUF * i
            for b in range(NBUF):
                drain_scatter(b)
                nxt = j0 + b + NBUF

                @pl.when(nxt < NCHUNK)
                def _pf():
                    issue(nxt, b)
            return _

        lax.fori_loop(0, NCHUNK // NBUF, group, None)
        for b in range(NCHUNK % NBUF):
            drain_scatter(b)
        plsc.subcore_barrier()
        pltpu.sync_copy(acc_sh.at[pl.ds(r0, nr)], out_hbm.at[c, pl.ds(r0, nr)])

    return _sc_prop


def _tc1_body(degp_ref, x_ref, w_ref, dis_ref, h1s_ref):
    dp = degp_ref[...]
    dis = lax.rsqrt(dp[0] + dp[1] + 2.0)
    h = jnp.dot(x_ref[...], w_ref[...], preferred_element_type=jnp.float32)
    dis_ref[...] = dis
    h1s_ref[...] = dis * h


def _tc1(degp, x, wcat1):
    return pl.pallas_call(
        _tc1_body,
        out_shape=(
            jax.ShapeDtypeStruct((N, 1), jnp.float32),
            jax.ShapeDtypeStruct((N, D1CAT), jnp.float32),
        ),
    )(degp, x, wcat1)


def _tc2_body(accp_ref, dis_ref, h1s_ref, b_ref, w_ref, h2s_ref):
    ap = accp_ref[...]
    dis = dis_ref[...]
    acc = ap[0] + ap[1] + 2.0 * h1s_ref[...]
    y1 = jnp.maximum(dis * acc + b_ref[...], 0.0)
    h2 = jnp.dot(y1, w_ref[...], preferred_element_type=jnp.float32)
    h2s_ref[...] = dis * h2


def _tc2(accp, dis, h1s, bcat1, wblk):
    return pl.pallas_call(
        _tc2_body,
        out_shape=jax.ShapeDtypeStruct((N, D2CAT), jnp.float32),
    )(accp, dis, h1s, bcat1, wblk)


_BB = 400
_NB = N // _BB


def _dot_t(a, b):
    return lax.dot_general(a, b, (((0,), (0,)), ((), ())),
                           preferred_element_type=jnp.float32)


def _tc3_body(accp_ref, dis_ref, h2s_ref, b_ref, bcol_ref,
              gmask_ref, wlin_ref, blin_ref, out_ref, pre_ref, xx_ref):
    ap = accp_ref[...]
    dis = dis_ref[...]
    o2 = dis * (ap[0] + ap[1] + 2.0 * h2s_ref[...]) + b_ref[...]

    g_row = lax.broadcasted_iota(jnp.int32, (1, G), 1)

    jj = lax.broadcasted_iota(jnp.int32, (32, FLAT), 1)
    rr = lax.broadcasted_iota(jnp.int32, (32, FLAT), 0)
    k_rep = (jj // 32 == rr).astype(jnp.float32)
    k_tile = (jj % 32 == rr).astype(jnp.float32)

    r64 = lax.broadcasted_iota(jnp.int32, (D2CAT, 32), 0)
    c32 = lax.broadcasted_iota(jnp.int32, (D2CAT, 32), 1)
    e1 = (r64 == c32).astype(jnp.float32)
    e2 = (r64 == c32 + 32).astype(jnp.float32)

    pre_ref[...] = jnp.dot(o2, e1, preferred_element_type=jnp.float32)
    xx_ref[...] = jnp.maximum(
        jnp.dot(o2, e2, preferred_element_type=jnp.float32), 0.0)

    bcol_full = bcol_ref[...]
    pre_full = pre_ref[...]
    rows = [
        jnp.max(jnp.where(bcol_full == g, pre_full, -jnp.inf),
                axis=0, keepdims=True)
        for g in range(G)
    ]
    m = jnp.concatenate(rows, axis=0)
    m = jnp.where(m == -jnp.inf, 0.0, m)

    def accbody(j, carry):
        s_acc, p_acc = carry
        pre_b = pre_ref[pl.ds(j * _BB, _BB), :]
        xx_b = xx_ref[pl.ds(j * _BB, _BB), :]
        s_ng = (bcol_ref[pl.ds(j * _BB, _BB), :] == g_row
                ).astype(jnp.float32)
        m_n = jnp.dot(s_ng, m, preferred_element_type=jnp.float32)
        e = jnp.exp(pre_b - m_n)
        s_acc = s_acc + _dot_t(s_ng, e)
        ek = jnp.dot(e, k_rep, preferred_element_type=jnp.float32)
        xt = jnp.dot(xx_b, k_tile, preferred_element_type=jnp.float32)
        p_acc = p_acc + _dot_t(s_ng, ek * xt)
        return s_acc, p_acc

    s_sum, p_sum = lax.fori_loop(
        0, _NB, accbody,
        (jnp.zeros((G, 32), jnp.float32), jnp.zeros((G, FLAT), jnp.float32)))

    srep = jnp.dot(s_sum, k_rep, preferred_element_type=jnp.float32)
    flat = p_sum / (srep + 1e-16) * gmask_ref[...]
    logits = jnp.dot(flat, wlin_ref[...],
                     preferred_element_type=jnp.float32) + blin_ref[...]
    z = logits - jnp.max(logits, axis=-1, keepdims=True)
    ez = jnp.exp(z)
    out_ref[...] = ez / jnp.sum(ez, axis=-1, keepdims=True)


def _tc3(accp, dis, h2s, bcat2, bcol, gmask, wlin_t, blin):
    return pl.pallas_call(
        _tc3_body,
        out_shape=jax.ShapeDtypeStruct((G, DOUT), jnp.float32),
        scratch_shapes=[pltpu.VMEM((N, 32), jnp.float32),
                        pltpu.VMEM((N, 32), jnp.float32)],
    )(accp, dis, h2s, bcat2, bcol, gmask, wlin_t, blin)


def kernel(x, edge_index, batch, num_graphs,
           Wa1, ba1, Wa2, ba2, Wx1, bx1, Wx2, bx2, Wlin, blin):
    src = edge_index[0].astype(jnp.int32)
    dst = edge_index[1].astype(jnp.int32)
    b32 = batch.astype(jnp.int32)

    wcat1 = jnp.concatenate([Wa1, Wx1], axis=1)
    bcat1 = jnp.concatenate([ba1, bx1])[None, :]
    wblk = jnp.zeros((D1CAT, D2CAT), jnp.float32)
    wblk = wblk.at[:64, :32].set(Wa2).at[64:, 32:].set(Wx2)
    bcat2 = jnp.concatenate([ba2, bx2])[None, :]
    gmask = (jnp.arange(G) < num_graphs).astype(jnp.float32)[:, None]

    zeros_n = jnp.zeros((N,), jnp.float32)
    zeros1 = jnp.zeros((N, D1CAT), jnp.float32)
    zeros2 = jnp.zeros((N, D2CAT), jnp.float32)

    degp = _get_sc_degree()(dst, zeros_n)
    dis, h1s = _tc1(degp.reshape(NCORES, N, 1), x, wcat1)
    acc1 = _make_sc_prop(D1CAT)(h1s, src, dst, zeros1)
    h2s = _tc2(acc1, dis, h1s, bcat1, wblk)
    acc2 = _make_sc_prop(D2CAT)(h2s, src, dst, zeros2)
    out = _tc3(acc2, dis, h2s, bcat2,
               b32[:, None], gmask, Wlin.T, blin[None, :])
    return out


# speedup vs baseline: 32.8733x; 1.1648x over previous
import functools

import jax
import jax.numpy as jnp
from jax import lax
from jax.experimental import pallas as pl
from jax.experimental.pallas import tpu as pltpu
from jax.experimental.pallas import tpu_sc as plsc

N = 10000
E = 320000
G = 64
D1CAT = 128
D2CAT = 64
DOUT = 10
FLAT = 32 * 32

NCORES = 2
NSUB = 16
E_PER_CORE = E // NCORES
E_PER_TILE = E_PER_CORE // NSUB
KCH = 80
NCHUNK = E_PER_TILE // KCH
NBUF = 3

def _row_range(s):
    r0 = s * 640
    nr = jnp.where(s == NSUB - 1, N - 640 * (NSUB - 1), 640)
    return r0, nr


@functools.cache
def _get_sc_degree():
    mesh = plsc.VectorSubcoreMesh(core_axis_name="c", subcore_axis_name="s")

    @functools.partial(
        pl.kernel,
        out_type=jax.ShapeDtypeStruct((NCORES, N), jnp.float32),
        mesh=mesh,
        scratch_types=[
            pltpu.VMEM((KCH,), jnp.int32),
            pltpu.VMEM((KCH,), jnp.int32),
            pltpu.VMEM((KCH,), jnp.float32),
            pltpu.SemaphoreType.DMA,
            pltpu.SemaphoreType.DMA,
            pltpu.VMEM_SHARED((N,), jnp.float32),
        ],
    )
    def _sc_degree(dst_hbm, zeros_hbm, out_hbm, idx0, idx1, ones_v,
                   sem0, sem1, acc_sh):
        c = lax.axis_index("c")
        s = lax.axis_index("s")
        bufs = ((idx0, sem0), (idx1, sem1))

        base = c * E_PER_CORE + s * E_PER_TILE

        def issue(j, b):
            iv, sm = bufs[b]
            pltpu.async_copy(dst_hbm.at[pl.ds(base + j * KCH, KCH)], iv, sm)

        issue(0, 0)
        issue(1, 1)

        for i in range(KCH // 16):
            ones_v[pl.ds(16 * i, 16)] = jnp.ones((16,), jnp.float32)

        @pl.when(s == 0)
        def _zero():
            pltpu.sync_copy(zeros_hbm, acc_sh)

        plsc.subcore_barrier()

        def pair(i, _):
            j0 = 2 * i
            for b in range(2):
                iv, sm = bufs[b]
                pltpu.make_async_copy(dst_hbm.at[pl.ds(base, KCH)], iv, sm).wait()
                pltpu.sync_copy(ones_v, acc_sh.at[iv], add=True)
                nxt = j0 + b + 2

                @pl.when(nxt < NCHUNK)
                def _pf():
                    issue(nxt, b)
            return _

        lax.fori_loop(0, NCHUNK // 2, pair, None)
        if NCHUNK % 2:
            iv, sm = bufs[0]
            pltpu.make_async_copy(dst_hbm.at[pl.ds(base, KCH)], iv, sm).wait()
            pltpu.sync_copy(ones_v, acc_sh.at[iv], add=True)
        plsc.subcore_barrier()

        @pl.when(s == 0)
        def _out():
            pltpu.sync_copy(acc_sh, out_hbm.at[c])

    return _sc_degree


@functools.cache
def _make_sc_prop(d):
    mesh = plsc.VectorSubcoreMesh(core_axis_name="c", subcore_axis_name="s")

    @functools.partial(
        pl.kernel,
        out_type=jax.ShapeDtypeStruct((NCORES, N, d), jnp.float32),
        mesh=mesh,
        compiler_params=pltpu.CompilerParams(use_tc_tiling_on_sc=(d == 128)),
        scratch_types=(
            [pltpu.VMEM((KCH,), jnp.int32)] * (2 * NBUF)
            + [pltpu.VMEM((KCH, d), jnp.float32)] * NBUF
            + [pltpu.SemaphoreType.DMA] * NBUF
            + [pltpu.VMEM_SHARED((N, d), jnp.float32)]
        ),
    )
    def _sc_prop(h_hbm, src_hbm, dst_hbm, zeros_hbm, out_hbm, *scratch):
        c = lax.axis_index("c")
        s = lax.axis_index("s")
        srcs = scratch[0:NBUF]
        dsts = scratch[NBUF:2 * NBUF]
        rows = scratch[2 * NBUF:3 * NBUF]
        sems = scratch[3 * NBUF:4 * NBUF]
        acc_sh = scratch[4 * NBUF]
        bufs = tuple(zip(srcs, dsts, rows, sems))

        base = c * E_PER_CORE + s * E_PER_TILE

        def issue(j, b):
            sv, dv, rv, sm = bufs[b]
            e0 = base + j * KCH
            pltpu.sync_copy(src_hbm.at[pl.ds(e0, KCH)], sv)
            pltpu.sync_copy(dst_hbm.at[pl.ds(e0, KCH)], dv)
            pltpu.async_copy(h_hbm.at[sv], rv, sm)

        def drain_scatter(b):
            sv, dv, rv, sm = bufs[b]
            pltpu.make_async_copy(h_hbm.at[sv], rv, sm).wait()
            pltpu.sync_copy(rv, acc_sh.at[dv], add=True)

        for b in range(NBUF):
            issue(b, b)
        r0, nr = _row_range(s)
        pltpu.sync_copy(zeros_hbm.at[pl.ds(r0, nr)], acc_sh.at[pl.ds(r0, nr)])
        plsc.subcore_barrier()

        def group(i, _):
            j0 = NBUF * i
            for b in range(NBUF):
                drain_scatter(b)
                nxt = j0 + b + NBUF

                @pl.when(nxt < NCHUNK)
                def _pf():
                    issue(nxt, b)
            return _

        lax.fori_loop(0, NCHUNK // NBUF, group, None)
        for b in range(NCHUNK % NBUF):
            drain_scatter(b)
        plsc.subcore_barrier()
        pltpu.sync_copy(acc_sh.at[pl.ds(r0, nr)], out_hbm.at[c, pl.ds(r0, nr)])

    return _sc_prop


def _tc1_body(degp_ref, x_ref, w_ref, dis_ref, h1s_ref):
    dp = degp_ref[...]
    dis = lax.rsqrt(dp[0] + dp[1] + 2.0)
    h = jnp.dot(x_ref[...], w_ref[...], preferred_element_type=jnp.float32)
    dis_ref[...] = dis
    h1s_ref[...] = dis * h


def _tc1(degp, x, wcat1):
    return pl.pallas_call(
        _tc1_body,
        out_shape=(
            jax.ShapeDtypeStruct((N, 1), jnp.float32),
            jax.ShapeDtypeStruct((N, D1CAT), jnp.float32),
        ),
    )(degp, x, wcat1)


def _tc2_body(accp_ref, dis_ref, h1s_ref, b_ref, w_ref, h2s_ref):
    ap = accp_ref[...]
    dis = dis_ref[...]
    acc = ap[0] + ap[1] + 2.0 * h1s_ref[...]
    y1 = jnp.maximum(dis * acc + b_ref[...], 0.0)
    h2 = jnp.dot(y1, w_ref[...], preferred_element_type=jnp.float32)
    h2s_ref[...] = dis * h2


def _tc2(accp, dis, h1s, bcat1, wblk):
    return pl.pallas_call(
        _tc2_body,
        out_shape=jax.ShapeDtypeStruct((N, D2CAT), jnp.float32),
    )(accp, dis, h1s, bcat1, wblk)


_BB = 400
_NB = N // _BB


def _dot_t(a, b):
    return lax.dot_general(a, b, (((0,), (0,)), ((), ())),
                           preferred_element_type=jnp.float32)


def _tc3_body(accp_ref, dis_ref, h2s_ref, b_ref, bcol_ref,
              gmask_ref, wlin_ref, blin_ref, out_ref, pre_ref, xx_ref):
    ap = accp_ref[...]
    dis = dis_ref[...]
    o2 = dis * (ap[0] + ap[1] + 2.0 * h2s_ref[...]) + b_ref[...]

    g_row = lax.broadcasted_iota(jnp.int32, (1, G), 1)

    jj = lax.broadcasted_iota(jnp.int32, (32, FLAT), 1)
    rr = lax.broadcasted_iota(jnp.int32, (32, FLAT), 0)
    k_rep = (jj // 32 == rr).astype(jnp.float32)
    k_tile = (jj % 32 == rr).astype(jnp.float32)

    r64 = lax.broadcasted_iota(jnp.int32, (D2CAT, 32), 0)
    c32 = lax.broadcasted_iota(jnp.int32, (D2CAT, 32), 1)
    e1 = (r64 == c32).astype(jnp.float32)
    e2 = (r64 == c32 + 32).astype(jnp.float32)

    pre_ref[...] = jnp.dot(o2, e1, preferred_element_type=jnp.float32)
    xx_ref[...] = jnp.maximum(
        jnp.dot(o2, e2, preferred_element_type=jnp.float32), 0.0)

    bcol_full = bcol_ref[...]
    pre_full = pre_ref[...]
    pspread = jnp.max(pre_full) - jnp.min(pre_full)
    bk = pspread + 1.0
    key = pre_full + bk * bcol_full.astype(jnp.float32)
    sh = 1
    while sh < N:
        shifted = jnp.concatenate(
            [jnp.full((sh, 32), -jnp.inf, jnp.float32), key[:N - sh]], axis=0)
        key = jnp.maximum(key, shifted)
        sh *= 2
    nxt = jnp.concatenate(
        [bcol_full[1:], jnp.full((1, 1), G, jnp.int32)], axis=0)
    sel_last = ((nxt != bcol_full)
                & (bcol_full == g_row)).astype(jnp.float32)
    gcolf = lax.broadcasted_iota(jnp.int32, (G, 1), 0).astype(jnp.float32)
    m = _dot_t(sel_last, key) - bk * gcolf

    def accbody(j, carry):
        s_acc, p_acc = carry
        pre_b = pre_ref[pl.ds(j * _BB, _BB), :]
        xx_b = xx_ref[pl.ds(j * _BB, _BB), :]
        s_ng = (bcol_ref[pl.ds(j * _BB, _BB), :] == g_row
                ).astype(jnp.float32)
        m_n = jnp.dot(s_ng, m, preferred_element_type=jnp.float32)
        e = jnp.exp(pre_b - m_n)
        s_acc = s_acc + _dot_t(s_ng, e)
        ek = jnp.dot(e, k_rep, preferred_element_type=jnp.float32)
        xt = jnp.dot(xx_b, k_tile, preferred_element_type=jnp.float32)
        p_acc = p_acc + _dot_t(s_ng, ek * xt)
        return s_acc, p_acc

    s_sum, p_sum = lax.fori_loop(
        0, _NB, accbody,
        (jnp.zeros((G, 32), jnp.float32), jnp.zeros((G, FLAT), jnp.float32)))

    srep = jnp.dot(s_sum, k_rep, preferred_element_type=jnp.float32)
    flat = p_sum / (srep + 1e-16) * gmask_ref[...]
    logits = jnp.dot(flat, wlin_ref[...],
                     preferred_element_type=jnp.float32) + blin_ref[...]
    z = logits - jnp.max(logits, axis=-1, keepdims=True)
    ez = jnp.exp(z)
    out_ref[...] = ez / jnp.sum(ez, axis=-1, keepdims=True)


def _tc3(accp, dis, h2s, bcat2, bcol, gmask, wlin_t, blin):
    return pl.pallas_call(
        _tc3_body,
        out_shape=jax.ShapeDtypeStruct((G, DOUT), jnp.float32),
        scratch_shapes=[pltpu.VMEM((N, 32), jnp.float32),
                        pltpu.VMEM((N, 32), jnp.float32)],
    )(accp, dis, h2s, bcat2, bcol, gmask, wlin_t, blin)


def kernel(x, edge_index, batch, num_graphs,
           Wa1, ba1, Wa2, ba2, Wx1, bx1, Wx2, bx2, Wlin, blin):
    src = edge_index[0].astype(jnp.int32)
    dst = edge_index[1].astype(jnp.int32)
    b32 = batch.astype(jnp.int32)

    wcat1 = jnp.concatenate([Wa1, Wx1], axis=1)
    bcat1 = jnp.concatenate([ba1, bx1])[None, :]
    wblk = jnp.zeros((D1CAT, D2CAT), jnp.float32)
    wblk = wblk.at[:64, :32].set(Wa2).at[64:, 32:].set(Wx2)
    bcat2 = jnp.concatenate([ba2, bx2])[None, :]
    gmask = (jnp.arange(G) < num_graphs).astype(jnp.float32)[:, None]

    zeros_n = jnp.zeros((N,), jnp.float32)
    zeros1 = jnp.zeros((N, D1CAT), jnp.float32)
    zeros2 = jnp.zeros((N, D2CAT), jnp.float32)

    degp = _get_sc_degree()(dst, zeros_n)
    dis, h1s = _tc1(degp.reshape(NCORES, N, 1), x, wcat1)
    acc1 = _make_sc_prop(D1CAT)(h1s, src, dst, zeros1)
    h2s = _tc2(acc1, dis, h1s, bcat1, wblk)
    acc2 = _make_sc_prop(D2CAT)(h2s, src, dst, zeros2)
    out = _tc3(acc2, dis, h2s, bcat2,
               b32[:, None], gmask, Wlin.T, blin[None, :])
    return out
